# Initial kernel scaffold; baseline (speedup 1.0000x reference)
import jax
import jax.numpy as jnp
from jax.experimental import pallas as pl


def kernel(x, W1, b1, W2, b2, W3, b3, E, W4, b4, W5, b5, W6, b6):
    raise NotImplementedError("write your pallas kernel here")


# speedup vs baseline: 4.9672x; 4.9672x over previous
import functools

import jax
import jax.numpy as jnp
from jax import lax
from jax.experimental import pallas as pl
from jax.experimental.pallas import tpu as pltpu
from jax.experimental.pallas import tpu_sc as plsc

_B = 16384
_NINPUT = 784
_NHIDDEN = 1024
_NLATENT = 32
_NEMB = 8192
_NEMBDIM = 32
_COMMIT = 0.25

_BT_ENC = 256
_BT_DEC = 512

_NC = 2
_NS = 16
_NW = _NC * _NS
_BPW = _B // _NW
_IDX_CHUNK = 128
_NCHUNK = _BPW // _IDX_CHUNK


def _enc_vq_body(x_ref, w1_ref, b1_ref, w2_ref, b2_ref, w3_ref, b3_ref,
                 e_ref, z_ref, idx_ref):
    h = jnp.dot(x_ref[...], w1_ref[...], preferred_element_type=jnp.float32)
    h = jnp.maximum(h + b1_ref[...], 0.0)
    h = jnp.dot(h, w2_ref[...], preferred_element_type=jnp.float32)
    h = jnp.maximum(h + b2_ref[...], 0.0)
    z = jnp.dot(h, w3_ref[...], preferred_element_type=jnp.float32) + b3_ref[...]
    z_ref[...] = z
    e = e_ref[...]
    s = lax.dot_general(z, e, (((1,), (1,)), ((), ())),
                        preferred_element_type=jnp.float32)
    e2 = jnp.sum(e * e, axis=1)
    score = s - e2[None, :]
    m = jnp.max(score, axis=1, keepdims=True)
    cols = lax.broadcasted_iota(jnp.int32, score.shape, 1)
    idx = jnp.min(jnp.where(score == m, cols, _NEMB), axis=1)
    idx_ref[...] = idx[:, None]


def _dec_body(q_ref, z_ref, w4_ref, b4_ref, w5_ref, b5_ref, w6_ref, b6_ref,
              out_ref, loss_ref):
    q = q_ref[...]
    z = z_ref[...]
    qst = z + (q - z)
    h = jnp.dot(qst, w4_ref[...], preferred_element_type=jnp.float32)
    h = jnp.maximum(h + b4_ref[...], 0.0)
    h = jnp.dot(h, w5_ref[...], preferred_element_type=jnp.float32)
    h = jnp.maximum(h + b5_ref[...], 0.0)
    o = jnp.dot(h, w6_ref[...], preferred_element_type=jnp.float32) + b6_ref[...]
    out_ref[...] = 1.0 / (1.0 + jnp.exp(-o))
    part = jnp.sum((q - z) ** 2, keepdims=True)[:1, :1]
    i = pl.program_id(0)
    tot = jnp.where(i == 0, part, loss_ref[...] + part)
    scale = (1.0 + _COMMIT) / (_B * _NLATENT)
    loss_ref[...] = jnp.where(i == pl.num_programs(0) - 1, tot * scale, tot)


@functools.cache
def _sc_gather_fn():
    mesh = plsc.VectorSubcoreMesh(core_axis_name="c", subcore_axis_name="s")

    @functools.partial(
        pl.kernel,
        out_type=jax.ShapeDtypeStruct((_B, _NEMBDIM), jnp.float32),
        mesh=mesh,
        scratch_types=[
            pltpu.VMEM((_NCHUNK, _IDX_CHUNK), jnp.int32),
            pltpu.VMEM((_BPW, _NEMBDIM), jnp.float32),
            pltpu.SemaphoreType.DMA,
        ],
        compiler_params=pltpu.CompilerParams(use_tc_tiling_on_sc=False),
    )
    def _sc_gather(table_hbm, idx_hbm, out_hbm, idx_v, rows_v, sem):
        wid = lax.axis_index("s") * _NC + lax.axis_index("c")
        pltpu.sync_copy(idx_hbm.at[wid], idx_v)
        cps = []
        for j in range(_NCHUNK):
            cps.append(pltpu.async_copy(
                table_hbm.at[idx_v.at[j]],
                rows_v.at[pl.ds(j * _IDX_CHUNK, _IDX_CHUNK)], sem))
        for cp in cps:
            cp.wait()
        pltpu.sync_copy(rows_v, out_hbm.at[pl.ds(wid * _BPW, _BPW)])

    return _sc_gather


def kernel(x, W1, b1, W2, b2, W3, b3, E, W4, b4, W5, b5, W6, b6):
    f32 = jnp.float32
    nb_enc = _B // _BT_ENC
    z, idx = pl.pallas_call(
        _enc_vq_body,
        grid=(nb_enc,),
        in_specs=[
            pl.BlockSpec((_BT_ENC, _NINPUT), lambda i: (i, 0)),
            pl.BlockSpec((_NINPUT, _NHIDDEN), lambda i: (0, 0)),
            pl.BlockSpec((1, _NHIDDEN), lambda i: (0, 0)),
            pl.BlockSpec((_NHIDDEN, _NHIDDEN), lambda i: (0, 0)),
            pl.BlockSpec((1, _NHIDDEN), lambda i: (0, 0)),
            pl.BlockSpec((_NHIDDEN, _NLATENT), lambda i: (0, 0)),
            pl.BlockSpec((1, _NLATENT), lambda i: (0, 0)),
            pl.BlockSpec((_NEMB, _NEMBDIM), lambda i: (0, 0)),
        ],
        out_specs=[
            pl.BlockSpec((_BT_ENC, _NLATENT), lambda i: (i, 0)),
            pl.BlockSpec((_BT_ENC, 1), lambda i: (i, 0)),
        ],
        out_shape=[
            jax.ShapeDtypeStruct((_B, _NLATENT), f32),
            jax.ShapeDtypeStruct((_B, 1), jnp.int32),
        ],
    )(x, W1.T, b1[None, :], W2.T, b2[None, :], W3.T, b3[None, :], E)

    q = _sc_gather_fn()(E, idx.reshape(_NW, _NCHUNK, _IDX_CHUNK))

    nb_dec = _B // _BT_DEC
    out, loss = pl.pallas_call(
        _dec_body,
        grid=(nb_dec,),
        in_specs=[
            pl.BlockSpec((_BT_DEC, _NEMBDIM), lambda i: (i, 0)),
            pl.BlockSpec((_BT_DEC, _NLATENT), lambda i: (i, 0)),
            pl.BlockSpec((_NLATENT, _NHIDDEN), lambda i: (0, 0)),
            pl.BlockSpec((1, _NHIDDEN), lambda i: (0, 0)),
            pl.BlockSpec((_NHIDDEN, _NHIDDEN), lambda i: (0, 0)),
            pl.BlockSpec((1, _NHIDDEN), lambda i: (0, 0)),
            pl.BlockSpec((_NHIDDEN, _NINPUT), lambda i: (0, 0)),
            pl.BlockSpec((1, _NINPUT), lambda i: (0, 0)),
        ],
        out_specs=[
            pl.BlockSpec((_BT_DEC, _NINPUT), lambda i: (i, 0)),
            pl.BlockSpec((1, 1), lambda i: (0, 0)),
        ],
        out_shape=[
            jax.ShapeDtypeStruct((_B, _NINPUT), f32),
            jax.ShapeDtypeStruct((1, 1), f32),
        ],
    )(q, z, W4.T, b4[None, :], W5.T, b5[None, :], W6.T, b6[None, :])
    return (out, loss.reshape(()))
